# baseline (speedup 1.0000x reference)
import functools

import jax
import jax.numpy as jnp
from jax import lax
from jax.experimental import pallas as pl
from jax.experimental.pallas import tpu as pltpu
from jax.experimental.pallas import tpu_sc as plsc

B = 16384
D = 32
V = 1000001
NC = 2
NS = 16
L = 16
NW = NC * NS
B_TC = 5120
B_SC = B - B_TC
BPW = B_SC // NW
NB = 16
K = 16


def _make_sc_kernel():
  mesh = plsc.VectorSubcoreMesh(core_axis_name="c", subcore_axis_name="s",
                                num_cores=NC, num_subcores=NS)

  @functools.partial(
      pl.kernel,
      out_type=jax.ShapeDtypeStruct((B_SC * D,), jnp.float32),
      mesh=mesh,
      scratch_types=[
          pltpu.VMEM((BPW + L,), jnp.int32),
          pltpu.VMEM((BPW * D,), jnp.float32),
          [pltpu.VMEM((D, 128), jnp.float32) for _ in range(NB)],
          pltpu.SemaphoreType.DMA,
      ],
      compiler_params=pltpu.CompilerParams(use_tc_tiling_on_sc=True,
                                           needs_layout_passes=False),
  )
  def gather_kernel(act_hbm, tt_hbm, out_hbm, idx_v, rows_v, ring, sem):
    wid = lax.axis_index("s") * NC + lax.axis_index("c")
    base = wid * BPW
    pltpu.sync_copy(act_hbm.at[pl.ds(base, BPW)], idx_v.at[pl.ds(0, BPW)])

    d_lo = lax.iota(jnp.int32, L)
    d_hi = d_lo + L
    half = NB // 2

    def fire(i, g):
      vv = idx_v[pl.ds(i * half, L)] + 1
      for j in range(half):
        r = vv[j]
        c128 = pl.multiple_of((r // 128) * 128, 128)
        pltpu.async_copy(tt_hbm.at[:, pl.ds(c128, 128)], ring[g * half + j],
                         sem)

    def extract(i, g):
      vv = idx_v[pl.ds(i * half, L)] + 1
      for j in range(half):
        pltpu.make_async_copy(tt_hbm.at[:, pl.ds(0, 128)],
                              ring[g * half + j], sem).wait()
        lane = jnp.full((L,), vv[j] % 128, jnp.int32)
        lo = plsc.load_gather(ring[g * half + j], [d_lo, lane])
        hi = plsc.load_gather(ring[g * half + j], [d_hi, lane])
        off = (i * half + j) * D
        rows_v[pl.ds(off, L)] = lo
        rows_v[pl.ds(off + L, L)] = hi

    nstep2 = BPW // half
    fire(0, 0)

    def step(k):
      fire(2 * k + 1, 1)
      extract(2 * k, 0)

      @pl.when(2 * k + 2 < nstep2)
      def _():
        fire(2 * k + 2, 0)

      extract(2 * k + 1, 1)

    pl.loop(0, nstep2 // 2)(step)
    pltpu.sync_copy(rows_v, out_hbm.at[pl.ds(base * D, BPW * D)])

  return gather_kernel


def _make_tc_kernel():
  nstep = B_TC // K

  def index_map_k(k):
    def im(i, idx_ref):
      return (0, (idx_ref[i * K + k] + 1) // 128)
    return im

  def body(idx_ref, *refs):
    blocks = refs[:K]
    out_ref = refs[K]
    i = pl.program_id(0)
    sub = lax.broadcasted_iota(jnp.int32, (128, K), 0)
    lanes = jnp.stack([(idx_ref[i * K + k] + 1) % 128 for k in range(K)])
    oh = (sub == lanes[None, :]).astype(jnp.float32)
    acc = jnp.zeros((D, K), jnp.float32)
    for k in range(K):
      sel = jnp.where(lax.broadcasted_iota(jnp.int32, (128, K), 1) == k,
                      oh, 0.0)
      acc = acc + jax.lax.dot_general(
          blocks[k][...], sel, (((1,), (0,)), ((), ())),
          precision=lax.Precision.HIGHEST,
          preferred_element_type=jnp.float32)
    out_ref[...] = acc.T

  grid_spec = pltpu.PrefetchScalarGridSpec(
      num_scalar_prefetch=1,
      grid=(nstep,),
      in_specs=[pl.BlockSpec((D, 128), index_map_k(k)) for k in range(K)],
      out_specs=pl.BlockSpec((K, D), lambda i, idx_ref: (i, 0)),
  )
  return pl.pallas_call(
      body,
      grid_spec=grid_spec,
      out_shape=jax.ShapeDtypeStruct((B_TC, D), jnp.float32),
  )


_sc_gather = _make_sc_kernel()
_tc_gather = _make_tc_kernel()


@jax.jit
def kernel(action, table):
  act = action.reshape(B)
  tt = table.T
  sc_flat = _sc_gather(act[:B_SC], tt)
  tc_rows = _tc_gather(act[B_SC:], *([tt] * K))
  return jnp.concatenate([sc_flat.reshape(B_SC, D), tc_rows], axis=0)


# speedup vs baseline: 1.6231x; 1.6231x over previous
import functools

import jax
import jax.numpy as jnp
from jax import lax
from jax.experimental import pallas as pl
from jax.experimental.pallas import tpu as pltpu
from jax.experimental.pallas import tpu_sc as plsc

B = 16384
D = 32
V = 1000001
NC = 2
NS = 16
L = 16
NW = NC * NS
B_TC = 5120
B_SC = B - B_TC
BPW = B_SC // NW
NB = 16
K = 16


def _make_sc_kernel():
  mesh = plsc.VectorSubcoreMesh(core_axis_name="c", subcore_axis_name="s",
                                num_cores=NC, num_subcores=NS)

  @functools.partial(
      pl.kernel,
      out_type=jax.ShapeDtypeStruct((B_SC * D,), jnp.float32),
      mesh=mesh,
      scratch_types=[
          pltpu.VMEM((BPW + L,), jnp.int32),
          pltpu.VMEM((BPW * D,), jnp.float32),
          [pltpu.VMEM((D, 128), jnp.float32) for _ in range(NB)],
          pltpu.SemaphoreType.DMA,
      ],
      compiler_params=pltpu.CompilerParams(use_tc_tiling_on_sc=True,
                                           needs_layout_passes=False),
  )
  def gather_kernel(act_hbm, tt_hbm, out_hbm, idx_v, rows_v, ring, sem):
    wid = lax.axis_index("s") * NC + lax.axis_index("c")
    base = wid * BPW
    pltpu.sync_copy(act_hbm.at[pl.ds(base, BPW)], idx_v.at[pl.ds(0, BPW)])

    d_lo = lax.iota(jnp.int32, L)
    d_hi = d_lo + L
    half = NB // 2

    def fire(i, g):
      vv = idx_v[pl.ds(i * half, L)] + 1
      for j in range(half):
        r = vv[j]
        c128 = pl.multiple_of((r // 128) * 128, 128)
        pltpu.async_copy(tt_hbm.at[:, pl.ds(c128, 128)], ring[g * half + j],
                         sem)

    def extract(i, g):
      vv = idx_v[pl.ds(i * half, L)] + 1
      for j in range(half):
        pltpu.make_async_copy(tt_hbm.at[:, pl.ds(0, 128)],
                              ring[g * half + j], sem).wait()
        lane = jnp.full((L,), vv[j] % 128, jnp.int32)
        lo = plsc.load_gather(ring[g * half + j], [d_lo, lane])
        hi = plsc.load_gather(ring[g * half + j], [d_hi, lane])
        off = (i * half + j) * D
        rows_v[pl.ds(off, L)] = lo
        rows_v[pl.ds(off + L, L)] = hi

    nstep2 = BPW // half
    fire(0, 0)

    def step(k):
      fire(2 * k + 1, 1)
      extract(2 * k, 0)

      @pl.when(2 * k + 2 < nstep2)
      def _():
        fire(2 * k + 2, 0)

      extract(2 * k + 1, 1)

    pl.loop(0, nstep2 // 2)(step)
    pltpu.sync_copy(rows_v, out_hbm.at[pl.ds(base * D, BPW * D)])

  return gather_kernel


def _make_tc_kernel():
  nstep = B_TC // K

  def index_map_k(k):
    def im(i, idx_ref):
      return (0, (idx_ref[i * K + k] + 1) // 128)
    return im

  def body(idx_ref, *refs):
    blocks = refs[:K]
    out_ref = refs[K]
    i = pl.program_id(0)
    cols = []
    for k in range(K):
      lane = (idx_ref[i * K + k] + 1) % 128
      cols.append(pltpu.roll(blocks[k][...], -lane, 1)[:, 0:1])
    out_ref[...] = jnp.concatenate(cols, axis=1).T

  grid_spec = pltpu.PrefetchScalarGridSpec(
      num_scalar_prefetch=1,
      grid=(nstep,),
      in_specs=[pl.BlockSpec((D, 128), index_map_k(k)) for k in range(K)],
      out_specs=pl.BlockSpec((K, D), lambda i, idx_ref: (i, 0)),
  )
  return pl.pallas_call(
      body,
      grid_spec=grid_spec,
      out_shape=jax.ShapeDtypeStruct((B_TC, D), jnp.float32),
  )


_sc_gather = _make_sc_kernel()
_tc_gather = _make_tc_kernel()


@jax.jit
def kernel(action, table):
  act = action.reshape(B)
  tt = table.T
  sc_flat = _sc_gather(act[:B_SC], tt)
  tc_rows = _tc_gather(act[B_SC:], *([tt] * K))
  return jnp.concatenate([sc_flat.reshape(B_SC, D), tc_rows], axis=0)


# speedup vs baseline: 4.0008x; 2.4649x over previous
import functools

import jax
import jax.numpy as jnp
from jax import lax
from jax.experimental import pallas as pl
from jax.experimental.pallas import tpu as pltpu
from jax.experimental.pallas import tpu_sc as plsc

B = 16384
D = 32
V = 1000001
NC = 2
NS = 16
L = 16
NW = NC * NS
BPW = B // NW
NB = 16
NSTEP = BPW // NB


def _make_kernel():
  mesh = plsc.VectorSubcoreMesh(core_axis_name="c", subcore_axis_name="s",
                                num_cores=NC, num_subcores=NS)

  @functools.partial(
      pl.kernel,
      out_type=jax.ShapeDtypeStruct((B * D,), jnp.float32),
      mesh=mesh,
      scratch_types=[
          pltpu.VMEM((BPW + L,), jnp.int32),
          pltpu.VMEM((BPW * D,), jnp.float32),
          [pltpu.VMEM((D, 128), jnp.float32) for _ in range(NB)],
          pltpu.SemaphoreType.DMA,
      ],
      compiler_params=pltpu.CompilerParams(use_tc_tiling_on_sc=True,
                                           needs_layout_passes=False),
  )
  def gather_kernel(act_hbm, tt_hbm, out_hbm, idx_v, rows_v, ring, sem):
    wid = lax.axis_index("s") * NC + lax.axis_index("c")
    base = wid * BPW
    pltpu.sync_copy(act_hbm.at[pl.ds(base, BPW)], idx_v.at[pl.ds(0, BPW)])

    d_lo = lax.iota(jnp.int32, L)
    d_hi = d_lo + L
    half = NB // 2

    def fire(i, g):
      vv = idx_v[pl.ds(i * half, L)] + 1
      for j in range(half):
        r = vv[j]
        c128 = pl.multiple_of((r // 128) * 128, 128)
        pltpu.async_copy(tt_hbm.at[:, pl.ds(c128, 128)], ring[g * half + j],
                         sem)

    def extract(i, g):
      vv = idx_v[pl.ds(i * half, L)] + 1
      for j in range(half):
        pltpu.make_async_copy(tt_hbm.at[:, pl.ds(0, 128)],
                              ring[g * half + j], sem).wait()
        lane = jnp.full((L,), vv[j] % 128, jnp.int32)
        lo = plsc.load_gather(ring[g * half + j], [d_lo, lane])
        hi = plsc.load_gather(ring[g * half + j], [d_hi, lane])
        off = (i * half + j) * D
        rows_v[pl.ds(off, L)] = lo
        rows_v[pl.ds(off + L, L)] = hi

    nstep2 = BPW // half
    fire(0, 0)

    def step(k):
      fire(2 * k + 1, 1)
      extract(2 * k, 0)

      @pl.when(2 * k + 2 < nstep2)
      def _():
        fire(2 * k + 2, 0)

      extract(2 * k + 1, 1)

    pl.loop(0, nstep2 // 2)(step)
    pltpu.sync_copy(rows_v, out_hbm.at[pl.ds(base * D, BPW * D)])

  return gather_kernel


_gather = _make_kernel()


@jax.jit
def kernel(action, table):
  act = action.reshape(B)
  flat = _gather(act, table.T)
  return flat.reshape(B, D)
